# baseline (speedup 1.0000x reference)
import jax
import jax.numpy as jnp
from jax import lax
from jax.experimental import pallas as pl
from jax.experimental.pallas import tpu as pltpu
from jax.experimental.pallas import tpu_sc as plsc

_VOCAB = 100000
_CTX = 2048
_D = 768
_BATCH = 4
_SEQ = 2048

_NC = 2
_NS = 16
_NW = _NC * _NS
_N = _BATCH * _SEQ
_PER_W = _N // _NW
_PP = _SEQ // _NW
_CP = 8
_C = _BATCH * _CP
_NCHUNK = _PP // _CP
_LANES = 16


def _body(src_hbm, pos_hbm, emb_hbm, out_hbm,
          idx_v, rows0, rows1, pos0, pos1,
          gsem0, gsem1, psem0, psem1, osem0, osem1):
    wid = lax.axis_index("s") * _NC + lax.axis_index("c")
    pos_base = wid * _PP

    rows_bufs = [rows0, rows1]
    pos_bufs = [pos0, pos1]
    gsems = [gsem0, gsem1]
    psems = [psem0, psem1]
    osems = [osem0, osem1]

    pltpu.sync_copy(src_hbm.at[wid], idx_v)

    def pos_slice(c):
        return pos_hbm.at[pl.ds(pos_base + c * _CP, _CP)]

    def issue(c):
        nb = c % 2
        pltpu.async_copy(emb_hbm.at[idx_v.at[c]], rows_bufs[nb], gsems[nb])
        pltpu.async_copy(pos_slice(c), pos_bufs[nb], psems[nb])

    def drain_stores(c):
        nb = c % 2
        for b in range(_BATCH):
            pltpu.make_async_copy(
                rows_bufs[nb].at[pl.ds(b * _CP, _CP)],
                out_hbm.at[pl.ds(b * _SEQ + pos_base + c * _CP, _CP)],
                osems[nb]).wait()

    issue(0)
    for c in range(_NCHUNK):
        nb = c % 2
        rows = rows_bufs[nb]
        pos = pos_bufs[nb]
        pltpu.make_async_copy(emb_hbm.at[idx_v.at[c]], rows,
                              gsems[nb]).wait()
        pltpu.make_async_copy(pos_slice(c), pos, psems[nb]).wait()
        if c + 1 < _NCHUNK:
            if c >= 1:
                drain_stores(c - 1)
            issue(c + 1)

        def p_body(p, carry):
            for j in range(_D // _LANES):
                s = pl.ds(j * _LANES, _LANES)
                pv = pos[p, s]
                for b in range(_BATCH):
                    plsc.addupdate(rows.at[b * _CP + p, s], pv)
            return carry

        lax.fori_loop(0, _CP, p_body, 0)

        for b in range(_BATCH):
            pltpu.async_copy(
                rows.at[pl.ds(b * _CP, _CP)],
                out_hbm.at[pl.ds(b * _SEQ + pos_base + c * _CP, _CP)],
                osems[nb])

    drain_stores(_NCHUNK - 2)
    drain_stores(_NCHUNK - 1)


@jax.jit
def _embed(src_t, emb_table, pos_table):
    kfn = pl.kernel(
        _body,
        out_type=jax.ShapeDtypeStruct((_N, _D), jnp.float32),
        mesh=plsc.VectorSubcoreMesh(core_axis_name="c", subcore_axis_name="s",
                                    num_cores=_NC, num_subcores=_NS),
        scratch_types=[
            pltpu.VMEM((_NCHUNK, _C), jnp.int32),
            pltpu.VMEM((_C, _D), jnp.float32),
            pltpu.VMEM((_C, _D), jnp.float32),
            pltpu.VMEM((_CP, _D), jnp.float32),
            pltpu.VMEM((_CP, _D), jnp.float32),
            pltpu.SemaphoreType.DMA,
            pltpu.SemaphoreType.DMA,
            pltpu.SemaphoreType.DMA,
            pltpu.SemaphoreType.DMA,
            pltpu.SemaphoreType.DMA,
            pltpu.SemaphoreType.DMA,
        ],
    )
    return kfn(src_t, pos_table, emb_table)


def kernel(src, emb_table, pos_table):
    batch, seq = src.shape
    s4 = src.reshape(_BATCH, _NW, _NCHUNK, _CP)
    src_t = (s4.transpose(1, 2, 0, 3)
             .reshape(_NW, _NCHUNK, _C).astype(jnp.int32))
    out = _embed(src_t, emb_table, pos_table)
    return out.reshape(batch, seq, _D)


# speedup vs baseline: 1.0187x; 1.0187x over previous
import jax
import jax.numpy as jnp
from jax import lax
from jax.experimental import pallas as pl
from jax.experimental.pallas import tpu as pltpu
from jax.experimental.pallas import tpu_sc as plsc

_VOCAB = 100000
_CTX = 2048
_D = 768
_BATCH = 4
_SEQ = 2048

_NC = 2
_NS = 16
_NW = _NC * _NS
_N = _BATCH * _SEQ
_PER_W = _N // _NW
_PP = _SEQ // _NW
_CP = 8
_C = _BATCH * _CP
_NCHUNK = _PP // _CP
_LANES = 16


def _body(src_hbm, pos_hbm, emb_hbm, out_hbm,
          idx_v, rows0, rows1, rows2, pos0, pos1, pos2,
          gsem0, gsem1, gsem2, psem0, psem1, psem2,
          osem0, osem1, osem2):
    wid = lax.axis_index("s") * _NC + lax.axis_index("c")
    pos_base = wid * _PP

    rows_bufs = [rows0, rows1, rows2]
    pos_bufs = [pos0, pos1, pos2]
    gsems = [gsem0, gsem1, gsem2]
    psems = [psem0, psem1, psem2]
    osems = [osem0, osem1, osem2]

    pltpu.sync_copy(src_hbm.at[wid], idx_v)

    def pos_slice(c):
        return pos_hbm.at[pl.ds(pos_base + c * _CP, _CP)]

    def issue(c):
        nb = c % 3
        pltpu.async_copy(emb_hbm.at[idx_v.at[c]], rows_bufs[nb], gsems[nb])
        pltpu.async_copy(pos_slice(c), pos_bufs[nb], psems[nb])

    def drain_stores(c):
        nb = c % 3
        for b in range(_BATCH):
            pltpu.make_async_copy(
                rows_bufs[nb].at[pl.ds(b * _CP, _CP)],
                out_hbm.at[pl.ds(b * _SEQ + pos_base + c * _CP, _CP)],
                osems[nb]).wait()

    issue(0)
    issue(1)
    for c in range(_NCHUNK):
        nb = c % 3
        rows = rows_bufs[nb]
        pos = pos_bufs[nb]
        pltpu.make_async_copy(emb_hbm.at[idx_v.at[c]], rows,
                              gsems[nb]).wait()
        pltpu.make_async_copy(pos_slice(c), pos, psems[nb]).wait()
        if c + 2 < _NCHUNK:
            if c >= 1:
                drain_stores(c - 1)
            issue(c + 2)

        def p_body(p, carry):
            for j in range(_D // _LANES):
                s = pl.ds(j * _LANES, _LANES)
                pv = pos[p, s]
                for b in range(_BATCH):
                    r = b * _CP
                    rows[r + p, s] = rows[r + p, s] + pv
            return carry

        lax.fori_loop(0, _CP, p_body, 0)

        for b in range(_BATCH):
            pltpu.async_copy(
                rows.at[pl.ds(b * _CP, _CP)],
                out_hbm.at[pl.ds(b * _SEQ + pos_base + c * _CP, _CP)],
                osems[nb])

    drain_stores(_NCHUNK - 3)
    drain_stores(_NCHUNK - 2)
    drain_stores(_NCHUNK - 1)


@jax.jit
def _embed(src_t, emb_table, pos_table):
    kfn = pl.kernel(
        _body,
        out_type=jax.ShapeDtypeStruct((_N, _D), jnp.float32),
        mesh=plsc.VectorSubcoreMesh(core_axis_name="c", subcore_axis_name="s",
                                    num_cores=_NC, num_subcores=_NS),
        scratch_types=[
            pltpu.VMEM((_NCHUNK, _C), jnp.int32),
            pltpu.VMEM((_C, _D), jnp.float32),
            pltpu.VMEM((_C, _D), jnp.float32),
            pltpu.VMEM((_C, _D), jnp.float32),
            pltpu.VMEM((_CP, _D), jnp.float32),
            pltpu.VMEM((_CP, _D), jnp.float32),
            pltpu.VMEM((_CP, _D), jnp.float32),
            pltpu.SemaphoreType.DMA,
            pltpu.SemaphoreType.DMA,
            pltpu.SemaphoreType.DMA,
            pltpu.SemaphoreType.DMA,
            pltpu.SemaphoreType.DMA,
            pltpu.SemaphoreType.DMA,
            pltpu.SemaphoreType.DMA,
            pltpu.SemaphoreType.DMA,
            pltpu.SemaphoreType.DMA,
        ],
    )
    return kfn(src_t, pos_table, emb_table)


def kernel(src, emb_table, pos_table):
    batch, seq = src.shape
    s4 = src.reshape(_BATCH, _NW, _NCHUNK, _CP)
    src_t = (s4.transpose(1, 2, 0, 3)
             .reshape(_NW, _NCHUNK, _C).astype(jnp.int32))
    out = _embed(src_t, emb_table, pos_table)
    return out.reshape(batch, seq, _D)


# speedup vs baseline: 1.0473x; 1.0281x over previous
import jax
import jax.numpy as jnp
from jax import lax
from jax.experimental import pallas as pl
from jax.experimental.pallas import tpu as pltpu
from jax.experimental.pallas import tpu_sc as plsc

_VOCAB = 100000
_CTX = 2048
_D = 768
_BATCH = 4
_SEQ = 2048

_NC = 2
_NS = 16
_NW = _NC * _NS
_N = _BATCH * _SEQ
_PER_W = _N // _NW
_PP = _SEQ // _NW
_CP = 8
_C = _BATCH * _CP
_NCHUNK = _PP // _CP
_LANES = 16


def _body(src_hbm, pos_hbm, emb_hbm, out_hbm,
          idx_v, rows0, rows1, rows2, rows3, pos0, pos1, pos2, pos3,
          gsem0, gsem1, gsem2, gsem3, psem0, psem1, psem2, psem3,
          osem0, osem1, osem2, osem3):
    wid = lax.axis_index("s") * _NC + lax.axis_index("c")
    pos_base = wid * _PP

    rows_bufs = [rows0, rows1, rows2, rows3]
    pos_bufs = [pos0, pos1, pos2, pos3]
    gsems = [gsem0, gsem1, gsem2, gsem3]
    psems = [psem0, psem1, psem2, psem3]
    osems = [osem0, osem1, osem2, osem3]

    pltpu.sync_copy(src_hbm.at[wid], idx_v)

    def pos_slice(c):
        return pos_hbm.at[pl.ds(pos_base + c * _CP, _CP)]

    def issue(c):
        nb = c % 4
        pltpu.async_copy(emb_hbm.at[idx_v.at[c]], rows_bufs[nb], gsems[nb])
        pltpu.async_copy(pos_slice(c), pos_bufs[nb], psems[nb])

    def drain_stores(c):
        nb = c % 4
        for b in range(_BATCH):
            pltpu.make_async_copy(
                rows_bufs[nb].at[pl.ds(b * _CP, _CP)],
                out_hbm.at[pl.ds(b * _SEQ + pos_base + c * _CP, _CP)],
                osems[nb]).wait()

    issue(0)
    issue(1)
    issue(2)
    for c in range(_NCHUNK):
        nb = c % 4
        rows = rows_bufs[nb]
        pos = pos_bufs[nb]
        pltpu.make_async_copy(emb_hbm.at[idx_v.at[c]], rows,
                              gsems[nb]).wait()
        pltpu.make_async_copy(pos_slice(c), pos, psems[nb]).wait()
        if c + 3 < _NCHUNK:
            if c >= 1:
                drain_stores(c - 1)
            issue(c + 3)

        def p_body(p, carry):
            for j in range(_D // _LANES):
                s = pl.ds(j * _LANES, _LANES)
                pv = pos[p, s]
                for b in range(_BATCH):
                    r = b * _CP
                    rows[r + p, s] = rows[r + p, s] + pv
            return carry

        lax.fori_loop(0, _CP, p_body, 0)

        for b in range(_BATCH):
            pltpu.async_copy(
                rows.at[pl.ds(b * _CP, _CP)],
                out_hbm.at[pl.ds(b * _SEQ + pos_base + c * _CP, _CP)],
                osems[nb])

    drain_stores(_NCHUNK - 4)
    drain_stores(_NCHUNK - 3)
    drain_stores(_NCHUNK - 2)
    drain_stores(_NCHUNK - 1)


@jax.jit
def _embed(src_t, emb_table, pos_table):
    kfn = pl.kernel(
        _body,
        out_type=jax.ShapeDtypeStruct((_N, _D), jnp.float32),
        mesh=plsc.VectorSubcoreMesh(core_axis_name="c", subcore_axis_name="s",
                                    num_cores=_NC, num_subcores=_NS),
        scratch_types=[
            pltpu.VMEM((_NCHUNK, _C), jnp.int32),
            pltpu.VMEM((_C, _D), jnp.float32),
            pltpu.VMEM((_C, _D), jnp.float32),
            pltpu.VMEM((_C, _D), jnp.float32),
            pltpu.VMEM((_C, _D), jnp.float32),
            pltpu.VMEM((_CP, _D), jnp.float32),
            pltpu.VMEM((_CP, _D), jnp.float32),
            pltpu.VMEM((_CP, _D), jnp.float32),
            pltpu.VMEM((_CP, _D), jnp.float32),
            pltpu.SemaphoreType.DMA,
            pltpu.SemaphoreType.DMA,
            pltpu.SemaphoreType.DMA,
            pltpu.SemaphoreType.DMA,
            pltpu.SemaphoreType.DMA,
            pltpu.SemaphoreType.DMA,
            pltpu.SemaphoreType.DMA,
            pltpu.SemaphoreType.DMA,
            pltpu.SemaphoreType.DMA,
            pltpu.SemaphoreType.DMA,
            pltpu.SemaphoreType.DMA,
            pltpu.SemaphoreType.DMA,
        ],
    )
    return kfn(src_t, pos_table, emb_table)


def kernel(src, emb_table, pos_table):
    batch, seq = src.shape
    s4 = src.reshape(_BATCH, _NW, _NCHUNK, _CP)
    src_t = (s4.transpose(1, 2, 0, 3)
             .reshape(_NW, _NCHUNK, _C).astype(jnp.int32))
    out = _embed(src_t, emb_table, pos_table)
    return out.reshape(batch, seq, _D)
